# baseline (speedup 1.0000x reference)
import functools

import jax
import jax.numpy as jnp
from jax import lax
from jax.experimental import pallas as pl
from jax.experimental.pallas import tpu as pltpu
from jax.experimental.pallas import tpu_sc as plsc

B, N = 32, 4096
NK = 16
K = 32
M1, M3 = 128, 32
NG = N // 16
SCAP = 64
BIG = 3.0e38
IBIG = 1 << 30
HI = lax.Precision.HIGHEST


def _lrelu(v):
    return jnp.where(v >= 0, v, v * jnp.float32(0.1))


def _relu(v):
    return jnp.maximum(v, 0.0)


def _dot(a, b, dims):
    return lax.dot_general(a, b, (dims, ((), ())),
                           preferred_element_type=jnp.float32, precision=HI)


def _d2_kernel(pts_ref, d2_ref):
    f32 = jnp.float32
    pts = pts_ref[0]
    c1 = pts[:M1, :]
    I3 = (lax.broadcasted_iota(jnp.int32, (3, 3), 0)
          == lax.broadcasted_iota(jnp.int32, (3, 3), 1)).astype(f32)
    ptsT = _dot(I3, pts, ((1,), (1,)))
    dx = c1[:, 0:1] - ptsT[0:1, :]
    dy = c1[:, 1:2] - ptsT[1:2, :]
    dz = c1[:, 2:3] - ptsT[2:3, :]
    d2 = dx * dx + dy * dy + dz * dz
    d2_ref[0] = d2


def _merge16(ak, av, bk, bv):
    bk = lax.rev(bk, (0,))
    bv = lax.rev(bv, (0,))
    m = ak <= bk
    lok = jnp.minimum(ak, bk)
    hik = jnp.maximum(ak, bk)
    lov = jnp.where(m, av, bv)
    hiv = jnp.where(m, bv, av)
    lok, lov = plsc.sort_key_val(lok, lov)
    hik, hiv = plsc.sort_key_val(hik, hiv)
    return lok, lov, hik, hiv


def _low32_of_sorted32s(a0k, a0v, a1k, a1v, b0k, b0v, b1k, b1v):
    rb0k, rb0v = lax.rev(b1k, (0,)), lax.rev(b1v, (0,))
    rb1k, rb1v = lax.rev(b0k, (0,)), lax.rev(b0v, (0,))
    m0 = a0k <= rb0k
    l0k = jnp.minimum(a0k, rb0k)
    l0v = jnp.where(m0, a0v, rb0v)
    m1 = a1k <= rb1k
    l1k = jnp.minimum(a1k, rb1k)
    l1v = jnp.where(m1, a1v, rb1v)
    mm = l0k <= l1k
    s0k = jnp.minimum(l0k, l1k)
    s0v = jnp.where(mm, l0v, l1v)
    s1k = jnp.maximum(l0k, l1k)
    s1v = jnp.where(mm, l1v, l0v)
    s0k, s0v = plsc.sort_key_val(s0k, s0v)
    s1k, s1v = plsc.sort_key_val(s1k, s1v)
    return s0k, s0v, s1k, s1v


def _make_sc_select(Bh, SPB):
    mesh = plsc.VectorSubcoreMesh(core_axis_name="c", subcore_axis_name="s")
    rows_per_sub = M1 // SPB

    @functools.partial(
        pl.kernel, mesh=mesh,
        out_type=(jax.ShapeDtypeStruct((Bh, M1, K * 16), jnp.float32),
                  jax.ShapeDtypeStruct((Bh, M3, K * M1), jnp.float32)),
        compiler_params=pltpu.CompilerParams(needs_layout_passes=False),
        scratch_types=[
            pltpu.VMEM((N * 16,), jnp.float32),
            pltpu.VMEM((N,), jnp.float32),
            pltpu.VMEM((N,), jnp.float32),
            pltpu.VMEM((K * 16,), jnp.float32),
            pltpu.VMEM((K * M1,), jnp.float32),
            pltpu.SemaphoreType.DMA,
            pltpu.SemaphoreType.DMA,
        ])
    def sc_select(d2_hbm, pay_hbm, out_hbm, sel3_hbm,
                  pay_v, d2a_v, d2b_v, g_v, oh_v, semA, semB):
        f32 = jnp.float32
        i32 = jnp.int32
        wid = lax.axis_index("s") * 2 + lax.axis_index("c")
        s = wid % Bh
        base = (wid // Bh) * rows_per_sub
        pltpu.sync_copy(pay_hbm.at[s], pay_v)
        iota16 = lax.broadcasted_iota(i32, (16,), 0)
        zeros16 = jnp.zeros((16,), f32)
        ones16 = jnp.ones((16,), f32)
        for i in range(K * M1 // 16):
            oh_v[pl.ds(i * 16, 16)] = zeros16

        def process_row(m, dbuf):
            k0, v0 = plsc.sort_key_val(dbuf[pl.ds(0, 16)], iota16)
            k1, v1 = plsc.sort_key_val(dbuf[pl.ds(16, 16)], iota16 + 16)
            c0k, c0v, c1k, c1v = _merge16(k0, v0, k1, v1)

            def chunk_body(ci, carry):
                c0k, c0v, c1k, c1v = carry
                ska, sva = plsc.sort_key_val(dbuf[pl.ds(ci * 32, 16)],
                                             iota16 + ci * 32)
                skb, svb = plsc.sort_key_val(dbuf[pl.ds(ci * 32 + 16, 16)],
                                             iota16 + ci * 32 + 16)
                a0k, a0v, a1k, a1v = _merge16(ska, sva, skb, svb)
                return _low32_of_sorted32s(c0k, c0v, c1k, c1v,
                                           a0k, a0v, a1k, a1v)

            f0k, s0v, f1k, s1v = lax.fori_loop(
                1, NG // 2, chunk_body, (c0k, c0v, c1k, c1v))

            @pl.when(m < M3)
            def _l3():
                e0k, e0v = plsc.sort_key_val(dbuf[pl.ds(0, 16)], iota16)
                e1k, e1v = plsc.sort_key_val(dbuf[pl.ds(16, 16)],
                                             iota16 + 16)
                t0k, t0v, t1k, t1v = _merge16(e0k, e0v, e1k, e1v)
                for ci in range(2, M1 // 16):
                    sk3, sv3 = plsc.sort_key_val(dbuf[pl.ds(ci * 16, 16)],
                                                 iota16 + ci * 16)
                    rsk3 = lax.rev(sk3, (0,))
                    rsv3 = lax.rev(sv3, (0,))
                    m13 = t1k <= rsk3
                    l1k3 = jnp.minimum(t1k, rsk3)
                    l1v3 = jnp.where(m13, t1v, rsv3)
                    mm3 = t0k <= l1k3
                    n0k3 = jnp.minimum(t0k, l1k3)
                    n0v3 = jnp.where(mm3, t0v, l1v3)
                    n1k3 = jnp.maximum(t0k, l1k3)
                    n1v3 = jnp.where(mm3, l1v3, t0v)
                    t0k, t0v = plsc.sort_key_val(n0k3, n0v3)
                    t1k, t1v = plsc.sort_key_val(n1k3, n1v3)
                plsc.store_scatter(oh_v, [iota16 * M1 + t0v], ones16)
                plsc.store_scatter(oh_v, [(iota16 + 16) * M1 + t1v], ones16)
                pltpu.sync_copy(oh_v, sel3_hbm.at[s, m])
                plsc.store_scatter(oh_v, [iota16 * M1 + t0v], zeros16)
                plsc.store_scatter(oh_v, [(iota16 + 16) * M1 + t1v], zeros16)

            for f in range(16):
                g0 = plsc.load_gather(pay_v, [s0v * 16 + f])
                plsc.store_scatter(g_v, [f * 32 + iota16], g0)
                g1 = plsc.load_gather(pay_v, [s1v * 16 + f])
                plsc.store_scatter(g_v, [f * 32 + 16 + iota16], g1)
            pltpu.sync_copy(g_v, out_hbm.at[s, m])

        pltpu.async_copy(d2_hbm.at[s, base], d2a_v, semA)

        def pair_body(i, carry):
            m0 = base + 2 * i
            m1 = base + 2 * i + 1
            pltpu.make_async_copy(d2_hbm.at[s, m0], d2a_v, semA).wait()
            pltpu.async_copy(d2_hbm.at[s, m1], d2b_v, semB)
            process_row(m0, d2a_v)
            pltpu.make_async_copy(d2_hbm.at[s, m1], d2b_v, semB).wait()

            @pl.when(i < rows_per_sub // 2 - 1)
            def _pref():
                pltpu.async_copy(d2_hbm.at[s, m0 + 2], d2a_v, semA)

            process_row(m1, d2b_v)
            return carry

        lax.fori_loop(0, rows_per_sub // 2, pair_body, jnp.int32(0))

    return sc_select


def _dense_kernel(g_ref, sel3_ref, pts_ref, Ws1s_ref, bs1s_ref,
                  Wp1p_ref, bp1_ref,
                  Ws3_ref, bs3_ref, Wp3_ref, bp3_ref,
                  Ws4_ref, bs4_ref, out_ref):
    f32 = jnp.float32
    pts = pts_ref[0]
    G = g_ref[0]
    c1 = pts[:M1, :]

    relx = G[:, 0:32] - c1[:, 0:1]
    rely = G[:, 32:64] - c1[:, 1:2]
    relz = G[:, 64:96] - c1[:, 2:3]
    wcols = []
    for n in range(NK):
        wn = _relu(relx * Ws1s_ref[0, n] + rely * Ws1s_ref[1, n]
                   + relz * Ws1s_ref[2, n] + bs1s_ref[n])
        wcols.append(wn)
    wflat = jnp.concatenate(wcols, axis=1)
    R = (lax.broadcasted_iota(jnp.int32, (NK * 32, NK), 0) // 32
         == lax.broadcasted_iota(jnp.int32, (NK * 32, NK), 1)).astype(f32)
    aggs = []
    for c in range(3):
        xc = G[:, (3 + c) * 32:(4 + c) * 32]
        xrep = jnp.concatenate([xc] * NK, axis=1)
        prod = wflat * xrep
        aggs.append(_dot(prod, R, ((1,), (0,))))
    feat1 = (_dot(aggs[0], Wp1p_ref[0], ((1,), (0,)))
             + _dot(aggs[1], Wp1p_ref[1], ((1,), (0,)))
             + _dot(aggs[2], Wp1p_ref[2], ((1,), (0,))))
    feat1 = feat1 * (1.0 / K) + bp1_ref[...]
    x1 = _lrelu(feat1)

    c3 = c1[:M3, :]
    P3 = jnp.concatenate([c1, x1], axis=1)
    SEL3 = sel3_ref[0]
    GG = _dot(SEL3, P3, ((1,), (0,)))
    RmT = (lax.broadcasted_iota(jnp.int32, (M3 * K, M3), 0) // K
           == lax.broadcasted_iota(jnp.int32, (M3 * K, M3), 1)).astype(f32)
    c3rep = _dot(RmT, c3, ((1,), (0,)))
    rel3 = GG[:, 0:3] - c3rep
    w3 = _relu(_dot(rel3, Ws3_ref[...], ((1,), (0,))) + bs3_ref[...])
    E = (lax.broadcasted_iota(jnp.int32, (NK, NK * 64), 1) // 64
         == lax.broadcasted_iota(jnp.int32, (NK, NK * 64), 0)).astype(f32)
    w_exp = _dot(w3, E, ((1,), (0,)))
    x_tile = jnp.concatenate([GG[:, 3:67]] * NK, axis=1)
    prod3 = w_exp * x_tile
    agg3 = jnp.sum(prod3.reshape(M3, K, NK * 64), axis=1)
    feat3 = _dot(agg3, Wp3_ref[...], ((1,), (0,))) * (1.0 / K) + bp3_ref[...]
    x3 = _lrelu(feat3)

    rel4 = c3 - c3[0:1, :]
    w4 = _relu(_dot(rel4, Ws4_ref[...], ((1,), (0,))) + bs4_ref[...])
    agg4 = _dot(w4, x3, ((0,), (0,))) * (1.0 / K)
    out_ref[0] = agg4


def _head_kernel(agg4_ref, Wp4_ref, bp4_ref, Wout_ref, Wreg_ref,
                 xout_ref, xreg_ref):
    v = _dot(agg4_ref[...], Wp4_ref[...], ((1,), (0,))) + bp4_ref[...]
    mu = jnp.mean(v, axis=0, keepdims=True)
    var = jnp.mean((v - mu) ** 2, axis=0, keepdims=True)
    vn = (v - mu) / jnp.sqrt(var + 1e-5)
    x4 = _lrelu(vn)
    xout = _dot(x4, Wout_ref[...], ((1,), (0,)))
    xout_ref[...] = xout
    xreg_ref[...] = _dot(_lrelu(xout), Wreg_ref[...], ((1,), (0,)))


def kernel(x, input_pts, Ws1, bs1, Wp1, bp1, Ws3, bs3, Wp3, bp3,
           Ws4, bs4, Wp4, bp4, Wout, Wreg):
    f32 = jnp.float32
    Wp1p = Wp1.reshape(NK, 3, 64).transpose(1, 0, 2)
    payload = jnp.concatenate(
        [input_pts, x, jnp.zeros((B, N, 10), f32)], axis=-1)
    payload = payload.reshape(B, N * 16)

    full = lambda s: pl.BlockSpec(s, lambda b: (0,) * len(s))
    whole = lambda s: pl.BlockSpec(s, lambda: (0,) * len(s))

    Bh = B // 4
    sc_half = _make_sc_select(Bh, B // Bh)

    def run_d2(ptsh):
        return pl.pallas_call(
            _d2_kernel,
            grid=(Bh,),
            in_specs=[pl.BlockSpec((1, N, 3), lambda b: (b, 0, 0))],
            out_specs=pl.BlockSpec((1, M1, N), lambda b: (b, 0, 0)),
            out_shape=jax.ShapeDtypeStruct((Bh, M1, N), f32),
        )(ptsh)

    def run_dense(Gh, SEL3h, ptsh):
        return pl.pallas_call(
            _dense_kernel,
            grid=(Bh,),
            in_specs=[
                pl.BlockSpec((1, M1, K * 16), lambda b: (b, 0, 0)),
                pl.BlockSpec((1, M3 * K, M1), lambda b: (b, 0, 0)),
                pl.BlockSpec((1, N, 3), lambda b: (b, 0, 0)),
                pl.BlockSpec(memory_space=pltpu.SMEM),
                pl.BlockSpec(memory_space=pltpu.SMEM),
                full((3, NK, 64)), full((64,)),
                full((3, NK)), full((NK,)), full((NK * 64, 192)),
                full((192,)), full((3, NK)), full((NK,)),
            ],
            out_specs=pl.BlockSpec((1, NK, 192), lambda b: (b, 0, 0)),
            out_shape=jax.ShapeDtypeStruct((Bh, NK, 192), f32),
        )(Gh, SEL3h, ptsh, Ws1, bs1, Wp1p, bp1, Ws3, bs3, Wp3, bp3,
          Ws4, bs4)

    parts = []
    for q in range(4):
        sl = slice(q * Bh, (q + 1) * Bh)
        d2q = run_d2(input_pts[sl])
        Gq, SELq = sc_half(d2q, payload[sl])
        parts.append((sl, Gq, SELq))
    aggs4 = [run_dense(Gq, SELq.reshape(Bh, M3 * K, M1), input_pts[sl])
             for sl, Gq, SELq in parts]
    agg4all = jnp.concatenate(aggs4, axis=0).reshape(B, NK * 192)

    xout, xreg = pl.pallas_call(
        _head_kernel,
        in_specs=[whole((B, NK * 192)), whole((NK * 192, 384)),
                  whole((384,)), whole((384, 256)), whole((256, 20))],
        out_specs=[whole((B, 256)), whole((B, 20))],
        out_shape=(jax.ShapeDtypeStruct((B, 256), f32),
                   jax.ShapeDtypeStruct((B, 20), f32)),
    )(agg4all, Wp4, bp4, Wout, Wreg)
    return (xout, xreg)


# speedup vs baseline: 1.1268x; 1.1268x over previous
import functools

import jax
import jax.numpy as jnp
from jax import lax
from jax.experimental import pallas as pl
from jax.experimental.pallas import tpu as pltpu
from jax.experimental.pallas import tpu_sc as plsc

B, N = 32, 4096
NK = 16
K = 32
M1, M3 = 128, 32
NG = N // 16
SCAP = 64
BIG = 3.0e38
IBIG = 1 << 30
HI = lax.Precision.HIGHEST


def _lrelu(v):
    return jnp.where(v >= 0, v, v * jnp.float32(0.1))


def _relu(v):
    return jnp.maximum(v, 0.0)


def _dot(a, b, dims):
    return lax.dot_general(a, b, (dims, ((), ())),
                           preferred_element_type=jnp.float32, precision=HI)


def _d2_kernel(pts_ref, d2_ref):
    f32 = jnp.float32
    pts = pts_ref[0]
    c1 = pts[:M1, :]
    I3 = (lax.broadcasted_iota(jnp.int32, (3, 3), 0)
          == lax.broadcasted_iota(jnp.int32, (3, 3), 1)).astype(f32)
    ptsT = _dot(I3, pts, ((1,), (1,)))
    dx = c1[:, 0:1] - ptsT[0:1, :]
    dy = c1[:, 1:2] - ptsT[1:2, :]
    dz = c1[:, 2:3] - ptsT[2:3, :]
    d2 = dx * dx + dy * dy + dz * dz
    d2_ref[0] = d2


def _merge16(ak, av, bk, bv):
    bk = lax.rev(bk, (0,))
    bv = lax.rev(bv, (0,))
    m = ak <= bk
    lok = jnp.minimum(ak, bk)
    hik = jnp.maximum(ak, bk)
    lov = jnp.where(m, av, bv)
    hiv = jnp.where(m, bv, av)
    lok, lov = plsc.sort_key_val(lok, lov)
    hik, hiv = plsc.sort_key_val(hik, hiv)
    return lok, lov, hik, hiv


def _low32_of_sorted32s(a0k, a0v, a1k, a1v, b0k, b0v, b1k, b1v):
    rb0k, rb0v = lax.rev(b1k, (0,)), lax.rev(b1v, (0,))
    rb1k, rb1v = lax.rev(b0k, (0,)), lax.rev(b0v, (0,))
    m0 = a0k <= rb0k
    l0k = jnp.minimum(a0k, rb0k)
    l0v = jnp.where(m0, a0v, rb0v)
    m1 = a1k <= rb1k
    l1k = jnp.minimum(a1k, rb1k)
    l1v = jnp.where(m1, a1v, rb1v)
    mm = l0k <= l1k
    s0k = jnp.minimum(l0k, l1k)
    s0v = jnp.where(mm, l0v, l1v)
    s1k = jnp.maximum(l0k, l1k)
    s1v = jnp.where(mm, l1v, l0v)
    s0k, s0v = plsc.sort_key_val(s0k, s0v)
    s1k, s1v = plsc.sort_key_val(s1k, s1v)
    return s0k, s0v, s1k, s1v


def _make_sc_select(Bh, SPB):
    mesh = plsc.VectorSubcoreMesh(core_axis_name="c", subcore_axis_name="s")
    rows_per_sub = M1 // SPB

    @functools.partial(
        pl.kernel, mesh=mesh,
        out_type=(jax.ShapeDtypeStruct((Bh, M1, K * 16), jnp.float32),
                  jax.ShapeDtypeStruct((Bh, M3, K * M1), jnp.float32)),
        compiler_params=pltpu.CompilerParams(needs_layout_passes=False),
        scratch_types=[
            pltpu.VMEM((N * 16,), jnp.float32),
            pltpu.VMEM((N,), jnp.float32),
            pltpu.VMEM((N,), jnp.float32),
            pltpu.VMEM((K * 16,), jnp.float32),
            pltpu.VMEM((K * M1,), jnp.float32),
            pltpu.SemaphoreType.DMA,
            pltpu.SemaphoreType.DMA,
        ])
    def sc_select(d2_hbm, pay_hbm, out_hbm, sel3_hbm,
                  pay_v, d2a_v, d2b_v, g_v, oh_v, semA, semB):
        f32 = jnp.float32
        i32 = jnp.int32
        wid = lax.axis_index("s") * 2 + lax.axis_index("c")
        s = wid % Bh
        base = (wid // Bh) * rows_per_sub
        pltpu.sync_copy(pay_hbm.at[s], pay_v)
        iota16 = lax.broadcasted_iota(i32, (16,), 0)
        zeros16 = jnp.zeros((16,), f32)
        ones16 = jnp.ones((16,), f32)
        for i in range(K * M1 // 16):
            oh_v[pl.ds(i * 16, 16)] = zeros16

        def process_row(m, dbuf):
            k0, v0 = plsc.sort_key_val(dbuf[pl.ds(0, 16)], iota16)
            k1, v1 = plsc.sort_key_val(dbuf[pl.ds(16, 16)], iota16 + 16)
            c0k, c0v, c1k, c1v = _merge16(k0, v0, k1, v1)

            def chunk_body(ci, carry):
                c0k, c0v, c1k, c1v = carry
                ska, sva = plsc.sort_key_val(dbuf[pl.ds(ci * 32, 16)],
                                             iota16 + ci * 32)
                skb, svb = plsc.sort_key_val(dbuf[pl.ds(ci * 32 + 16, 16)],
                                             iota16 + ci * 32 + 16)
                a0k, a0v, a1k, a1v = _merge16(ska, sva, skb, svb)
                return _low32_of_sorted32s(c0k, c0v, c1k, c1v,
                                           a0k, a0v, a1k, a1v)

            f0k, s0v, f1k, s1v = lax.fori_loop(
                1, NG // 2, chunk_body, (c0k, c0v, c1k, c1v))

            @pl.when(m < M3)
            def _l3():
                e0k, e0v = plsc.sort_key_val(dbuf[pl.ds(0, 16)], iota16)
                e1k, e1v = plsc.sort_key_val(dbuf[pl.ds(16, 16)],
                                             iota16 + 16)
                t0k, t0v, t1k, t1v = _merge16(e0k, e0v, e1k, e1v)
                for ci in range(2, M1 // 16):
                    sk3, sv3 = plsc.sort_key_val(dbuf[pl.ds(ci * 16, 16)],
                                                 iota16 + ci * 16)
                    rsk3 = lax.rev(sk3, (0,))
                    rsv3 = lax.rev(sv3, (0,))
                    m13 = t1k <= rsk3
                    l1k3 = jnp.minimum(t1k, rsk3)
                    l1v3 = jnp.where(m13, t1v, rsv3)
                    mm3 = t0k <= l1k3
                    n0k3 = jnp.minimum(t0k, l1k3)
                    n0v3 = jnp.where(mm3, t0v, l1v3)
                    n1k3 = jnp.maximum(t0k, l1k3)
                    n1v3 = jnp.where(mm3, l1v3, t0v)
                    t0k, t0v = plsc.sort_key_val(n0k3, n0v3)
                    t1k, t1v = plsc.sort_key_val(n1k3, n1v3)
                plsc.store_scatter(oh_v, [iota16 * M1 + t0v], ones16)
                plsc.store_scatter(oh_v, [(iota16 + 16) * M1 + t1v], ones16)
                pltpu.sync_copy(oh_v, sel3_hbm.at[s, m])
                plsc.store_scatter(oh_v, [iota16 * M1 + t0v], zeros16)
                plsc.store_scatter(oh_v, [(iota16 + 16) * M1 + t1v], zeros16)

            for f in range(16):
                g0 = plsc.load_gather(pay_v, [s0v * 16 + f])
                plsc.store_scatter(g_v, [f * 32 + iota16], g0)
                g1 = plsc.load_gather(pay_v, [s1v * 16 + f])
                plsc.store_scatter(g_v, [f * 32 + 16 + iota16], g1)
            pltpu.sync_copy(g_v, out_hbm.at[s, m])

        pltpu.async_copy(d2_hbm.at[s, base], d2a_v, semA)

        def pair_body(i, carry):
            m0 = base + 2 * i
            m1 = base + 2 * i + 1
            pltpu.make_async_copy(d2_hbm.at[s, m0], d2a_v, semA).wait()
            pltpu.async_copy(d2_hbm.at[s, m1], d2b_v, semB)
            process_row(m0, d2a_v)
            pltpu.make_async_copy(d2_hbm.at[s, m1], d2b_v, semB).wait()

            @pl.when(i < rows_per_sub // 2 - 1)
            def _pref():
                pltpu.async_copy(d2_hbm.at[s, m0 + 2], d2a_v, semA)

            process_row(m1, d2b_v)
            return carry

        lax.fori_loop(0, rows_per_sub // 2, pair_body, jnp.int32(0))

    return sc_select


def _dense_kernel(g_ref, sel3_ref, pts_ref, Ws1s_ref, bs1s_ref,
                  Wp1p_ref, bp1_ref,
                  Ws3_ref, bs3_ref, Wp3_ref, bp3_ref,
                  Ws4_ref, bs4_ref, out_ref):
    f32 = jnp.float32
    pts = pts_ref[0]
    G = g_ref[0]
    c1 = pts[:M1, :]

    relx = G[:, 0:32] - c1[:, 0:1]
    rely = G[:, 32:64] - c1[:, 1:2]
    relz = G[:, 64:96] - c1[:, 2:3]
    wcols = []
    for n in range(NK):
        wn = _relu(relx * Ws1s_ref[0, n] + rely * Ws1s_ref[1, n]
                   + relz * Ws1s_ref[2, n] + bs1s_ref[n])
        wcols.append(wn)
    wflat = jnp.concatenate(wcols, axis=1)
    R = (lax.broadcasted_iota(jnp.int32, (NK * 32, NK), 0) // 32
         == lax.broadcasted_iota(jnp.int32, (NK * 32, NK), 1)).astype(f32)
    aggs = []
    for c in range(3):
        xc = G[:, (3 + c) * 32:(4 + c) * 32]
        xrep = jnp.concatenate([xc] * NK, axis=1)
        prod = wflat * xrep
        aggs.append(_dot(prod, R, ((1,), (0,))))
    feat1 = (_dot(aggs[0], Wp1p_ref[0], ((1,), (0,)))
             + _dot(aggs[1], Wp1p_ref[1], ((1,), (0,)))
             + _dot(aggs[2], Wp1p_ref[2], ((1,), (0,))))
    feat1 = feat1 * (1.0 / K) + bp1_ref[...]
    x1 = _lrelu(feat1)

    c3 = c1[:M3, :]
    P3 = jnp.concatenate([c1, x1], axis=1)
    SEL3 = sel3_ref[0]
    GG = _dot(SEL3, P3, ((1,), (0,)))
    RmT = (lax.broadcasted_iota(jnp.int32, (M3 * K, M3), 0) // K
           == lax.broadcasted_iota(jnp.int32, (M3 * K, M3), 1)).astype(f32)
    c3rep = _dot(RmT, c3, ((1,), (0,)))
    rel3 = GG[:, 0:3] - c3rep
    w3 = _relu(_dot(rel3, Ws3_ref[...], ((1,), (0,))) + bs3_ref[...])
    E = (lax.broadcasted_iota(jnp.int32, (NK, NK * 64), 1) // 64
         == lax.broadcasted_iota(jnp.int32, (NK, NK * 64), 0)).astype(f32)
    w_exp = _dot(w3, E, ((1,), (0,)))
    x_tile = jnp.concatenate([GG[:, 3:67]] * NK, axis=1)
    prod3 = w_exp * x_tile
    agg3 = jnp.sum(prod3.reshape(M3, K, NK * 64), axis=1)
    feat3 = _dot(agg3, Wp3_ref[...], ((1,), (0,))) * (1.0 / K) + bp3_ref[...]
    x3 = _lrelu(feat3)

    rel4 = c3 - c3[0:1, :]
    w4 = _relu(_dot(rel4, Ws4_ref[...], ((1,), (0,))) + bs4_ref[...])
    agg4 = _dot(w4, x3, ((0,), (0,))) * (1.0 / K)
    out_ref[0] = agg4


def _head_kernel(agg4_ref, Wp4_ref, bp4_ref, Wout_ref, Wreg_ref,
                 xout_ref, xreg_ref):
    v = _dot(agg4_ref[...], Wp4_ref[...], ((1,), (0,))) + bp4_ref[...]
    mu = jnp.mean(v, axis=0, keepdims=True)
    var = jnp.mean((v - mu) ** 2, axis=0, keepdims=True)
    vn = (v - mu) / jnp.sqrt(var + 1e-5)
    x4 = _lrelu(vn)
    xout = _dot(x4, Wout_ref[...], ((1,), (0,)))
    xout_ref[...] = xout
    xreg_ref[...] = _dot(_lrelu(xout), Wreg_ref[...], ((1,), (0,)))


def kernel(x, input_pts, Ws1, bs1, Wp1, bp1, Ws3, bs3, Wp3, bp3,
           Ws4, bs4, Wp4, bp4, Wout, Wreg):
    f32 = jnp.float32
    Wp1p = Wp1.reshape(NK, 3, 64).transpose(1, 0, 2)
    payload = jnp.concatenate(
        [input_pts, x, jnp.zeros((B, N, 10), f32)], axis=-1)
    payload = payload.reshape(B, N * 16)

    full = lambda s: pl.BlockSpec(s, lambda b: (0,) * len(s))
    whole = lambda s: pl.BlockSpec(s, lambda: (0,) * len(s))

    Bh = B // 2
    sc_half = _make_sc_select(Bh, B // Bh)

    def run_d2(ptsh):
        return pl.pallas_call(
            _d2_kernel,
            grid=(Bh,),
            in_specs=[pl.BlockSpec((1, N, 3), lambda b: (b, 0, 0))],
            out_specs=pl.BlockSpec((1, M1, N), lambda b: (b, 0, 0)),
            out_shape=jax.ShapeDtypeStruct((Bh, M1, N), f32),
        )(ptsh)

    def run_dense(Gh, SEL3h, ptsh):
        return pl.pallas_call(
            _dense_kernel,
            grid=(Bh,),
            in_specs=[
                pl.BlockSpec((1, M1, K * 16), lambda b: (b, 0, 0)),
                pl.BlockSpec((1, M3 * K, M1), lambda b: (b, 0, 0)),
                pl.BlockSpec((1, N, 3), lambda b: (b, 0, 0)),
                pl.BlockSpec(memory_space=pltpu.SMEM),
                pl.BlockSpec(memory_space=pltpu.SMEM),
                full((3, NK, 64)), full((64,)),
                full((3, NK)), full((NK,)), full((NK * 64, 192)),
                full((192,)), full((3, NK)), full((NK,)),
            ],
            out_specs=pl.BlockSpec((1, NK, 192), lambda b: (b, 0, 0)),
            out_shape=jax.ShapeDtypeStruct((Bh, NK, 192), f32),
        )(Gh, SEL3h, ptsh, Ws1, bs1, Wp1p, bp1, Ws3, bs3, Wp3, bp3,
          Ws4, bs4)

    parts = []
    for q in range(2):
        sl = slice(q * Bh, (q + 1) * Bh)
        d2q = run_d2(input_pts[sl])
        Gq, SELq = sc_half(d2q, payload[sl])
        parts.append((sl, Gq, SELq))
    aggs4 = [run_dense(Gq, SELq.reshape(Bh, M3 * K, M1), input_pts[sl])
             for sl, Gq, SELq in parts]
    agg4all = jnp.concatenate(aggs4, axis=0).reshape(B, NK * 192)

    xout, xreg = pl.pallas_call(
        _head_kernel,
        in_specs=[whole((B, NK * 192)), whole((NK * 192, 384)),
                  whole((384,)), whole((384, 256)), whole((256, 20))],
        out_specs=[whole((B, 256)), whole((B, 20))],
        out_shape=(jax.ShapeDtypeStruct((B, 256), f32),
                   jax.ShapeDtypeStruct((B, 20), f32)),
    )(agg4all, Wp4, bp4, Wout, Wreg)
    return (xout, xreg)


# speedup vs baseline: 1.1412x; 1.0128x over previous
import functools

import jax
import jax.numpy as jnp
from jax import lax
from jax.experimental import pallas as pl
from jax.experimental.pallas import tpu as pltpu
from jax.experimental.pallas import tpu_sc as plsc

B, N = 32, 4096
NK = 16
K = 32
M1, M3 = 128, 32
NG = N // 16
SCAP = 64
BIG = 3.0e38
IBIG = 1 << 30
HI = lax.Precision.HIGHEST


def _lrelu(v):
    return jnp.where(v >= 0, v, v * jnp.float32(0.1))


def _relu(v):
    return jnp.maximum(v, 0.0)


def _dot(a, b, dims):
    return lax.dot_general(a, b, (dims, ((), ())),
                           preferred_element_type=jnp.float32, precision=HI)


def _d2_kernel(pts_ref, ptsT_ref, d2_ref):
    pts = pts_ref[0]
    c1 = pts[:M1, :]
    ptsT = ptsT_ref[0]
    dx = c1[:, 0:1] - ptsT[0:1, :]
    dy = c1[:, 1:2] - ptsT[1:2, :]
    dz = c1[:, 2:3] - ptsT[2:3, :]
    d2 = dx * dx + dy * dy + dz * dz
    d2_ref[0] = d2


def _merge16(ak, av, bk, bv):
    bk = lax.rev(bk, (0,))
    bv = lax.rev(bv, (0,))
    m = ak <= bk
    lok = jnp.minimum(ak, bk)
    hik = jnp.maximum(ak, bk)
    lov = jnp.where(m, av, bv)
    hiv = jnp.where(m, bv, av)
    lok, lov = plsc.sort_key_val(lok, lov)
    hik, hiv = plsc.sort_key_val(hik, hiv)
    return lok, lov, hik, hiv


def _low32_of_sorted32s(a0k, a0v, a1k, a1v, b0k, b0v, b1k, b1v):
    rb0k, rb0v = lax.rev(b1k, (0,)), lax.rev(b1v, (0,))
    rb1k, rb1v = lax.rev(b0k, (0,)), lax.rev(b0v, (0,))
    m0 = a0k <= rb0k
    l0k = jnp.minimum(a0k, rb0k)
    l0v = jnp.where(m0, a0v, rb0v)
    m1 = a1k <= rb1k
    l1k = jnp.minimum(a1k, rb1k)
    l1v = jnp.where(m1, a1v, rb1v)
    mm = l0k <= l1k
    s0k = jnp.minimum(l0k, l1k)
    s0v = jnp.where(mm, l0v, l1v)
    s1k = jnp.maximum(l0k, l1k)
    s1v = jnp.where(mm, l1v, l0v)
    s0k, s0v = plsc.sort_key_val(s0k, s0v)
    s1k, s1v = plsc.sort_key_val(s1k, s1v)
    return s0k, s0v, s1k, s1v


def _make_sc_select(Bh, SPB):
    mesh = plsc.VectorSubcoreMesh(core_axis_name="c", subcore_axis_name="s")
    rows_per_sub = M1 // SPB

    @functools.partial(
        pl.kernel, mesh=mesh,
        out_type=(jax.ShapeDtypeStruct((Bh, M1, K * 16), jnp.float32),
                  jax.ShapeDtypeStruct((Bh, M3, K * M1), jnp.float32)),
        compiler_params=pltpu.CompilerParams(needs_layout_passes=False),
        scratch_types=[
            pltpu.VMEM((N * 16,), jnp.float32),
            pltpu.VMEM((N,), jnp.float32),
            pltpu.VMEM((N,), jnp.float32),
            pltpu.VMEM((K * 16,), jnp.float32),
            pltpu.VMEM((K * M1,), jnp.float32),
            pltpu.SemaphoreType.DMA,
            pltpu.SemaphoreType.DMA,
        ])
    def sc_select(d2_hbm, pay_hbm, out_hbm, sel3_hbm,
                  pay_v, d2a_v, d2b_v, g_v, oh_v, semA, semB):
        f32 = jnp.float32
        i32 = jnp.int32
        wid = lax.axis_index("s") * 2 + lax.axis_index("c")
        s = wid % Bh
        base = (wid // Bh) * rows_per_sub
        pltpu.sync_copy(pay_hbm.at[s], pay_v)
        iota16 = lax.broadcasted_iota(i32, (16,), 0)
        zeros16 = jnp.zeros((16,), f32)
        ones16 = jnp.ones((16,), f32)
        for i in range(K * M1 // 16):
            oh_v[pl.ds(i * 16, 16)] = zeros16

        def process_row(m, dbuf):
            k0, v0 = plsc.sort_key_val(dbuf[pl.ds(0, 16)], iota16)
            k1, v1 = plsc.sort_key_val(dbuf[pl.ds(16, 16)], iota16 + 16)
            c0k, c0v, c1k, c1v = _merge16(k0, v0, k1, v1)

            def chunk_body(ci, carry):
                c0k, c0v, c1k, c1v = carry
                ska, sva = plsc.sort_key_val(dbuf[pl.ds(ci * 32, 16)],
                                             iota16 + ci * 32)
                skb, svb = plsc.sort_key_val(dbuf[pl.ds(ci * 32 + 16, 16)],
                                             iota16 + ci * 32 + 16)
                a0k, a0v, a1k, a1v = _merge16(ska, sva, skb, svb)
                return _low32_of_sorted32s(c0k, c0v, c1k, c1v,
                                           a0k, a0v, a1k, a1v)

            f0k, s0v, f1k, s1v = lax.fori_loop(
                1, NG // 2, chunk_body, (c0k, c0v, c1k, c1v))

            @pl.when(m < M3)
            def _l3():
                e0k, e0v = plsc.sort_key_val(dbuf[pl.ds(0, 16)], iota16)
                e1k, e1v = plsc.sort_key_val(dbuf[pl.ds(16, 16)],
                                             iota16 + 16)
                t0k, t0v, t1k, t1v = _merge16(e0k, e0v, e1k, e1v)
                for ci in range(2, M1 // 16):
                    sk3, sv3 = plsc.sort_key_val(dbuf[pl.ds(ci * 16, 16)],
                                                 iota16 + ci * 16)
                    rsk3 = lax.rev(sk3, (0,))
                    rsv3 = lax.rev(sv3, (0,))
                    m13 = t1k <= rsk3
                    l1k3 = jnp.minimum(t1k, rsk3)
                    l1v3 = jnp.where(m13, t1v, rsv3)
                    mm3 = t0k <= l1k3
                    n0k3 = jnp.minimum(t0k, l1k3)
                    n0v3 = jnp.where(mm3, t0v, l1v3)
                    n1k3 = jnp.maximum(t0k, l1k3)
                    n1v3 = jnp.where(mm3, l1v3, t0v)
                    t0k, t0v = plsc.sort_key_val(n0k3, n0v3)
                    t1k, t1v = plsc.sort_key_val(n1k3, n1v3)
                plsc.store_scatter(oh_v, [iota16 * M1 + t0v], ones16)
                plsc.store_scatter(oh_v, [(iota16 + 16) * M1 + t1v], ones16)
                pltpu.sync_copy(oh_v, sel3_hbm.at[s, m])
                plsc.store_scatter(oh_v, [iota16 * M1 + t0v], zeros16)
                plsc.store_scatter(oh_v, [(iota16 + 16) * M1 + t1v], zeros16)

            for f in range(16):
                g0 = plsc.load_gather(pay_v, [s0v * 16 + f])
                plsc.store_scatter(g_v, [f * 32 + iota16], g0)
                g1 = plsc.load_gather(pay_v, [s1v * 16 + f])
                plsc.store_scatter(g_v, [f * 32 + 16 + iota16], g1)
            pltpu.sync_copy(g_v, out_hbm.at[s, m])

        pltpu.async_copy(d2_hbm.at[s, base], d2a_v, semA)

        def pair_body(i, carry):
            m0 = base + 2 * i
            m1 = base + 2 * i + 1
            pltpu.make_async_copy(d2_hbm.at[s, m0], d2a_v, semA).wait()
            pltpu.async_copy(d2_hbm.at[s, m1], d2b_v, semB)
            process_row(m0, d2a_v)
            pltpu.make_async_copy(d2_hbm.at[s, m1], d2b_v, semB).wait()

            @pl.when(i < rows_per_sub // 2 - 1)
            def _pref():
                pltpu.async_copy(d2_hbm.at[s, m0 + 2], d2a_v, semA)

            process_row(m1, d2b_v)
            return carry

        lax.fori_loop(0, rows_per_sub // 2, pair_body, jnp.int32(0))

    return sc_select


def _dense_kernel(g_ref, sel3_ref, pts_ref, Ws1s_ref, bs1s_ref,
                  Wp1p_ref, bp1_ref,
                  Ws3_ref, bs3_ref, Wp3_ref, bp3_ref,
                  Ws4_ref, bs4_ref, out_ref):
    f32 = jnp.float32
    pts = pts_ref[0]
    G = g_ref[0]
    c1 = pts[:M1, :]

    relx = G[:, 0:32] - c1[:, 0:1]
    rely = G[:, 32:64] - c1[:, 1:2]
    relz = G[:, 64:96] - c1[:, 2:3]
    wcols = []
    for n in range(NK):
        wn = _relu(relx * Ws1s_ref[0, n] + rely * Ws1s_ref[1, n]
                   + relz * Ws1s_ref[2, n] + bs1s_ref[n])
        wcols.append(wn)
    wflat = jnp.concatenate(wcols, axis=1)
    R = (lax.broadcasted_iota(jnp.int32, (NK * 32, NK), 0) // 32
         == lax.broadcasted_iota(jnp.int32, (NK * 32, NK), 1)).astype(f32)
    aggs = []
    for c in range(3):
        xc = G[:, (3 + c) * 32:(4 + c) * 32]
        xrep = jnp.concatenate([xc] * NK, axis=1)
        prod = wflat * xrep
        aggs.append(_dot(prod, R, ((1,), (0,))))
    feat1 = (_dot(aggs[0], Wp1p_ref[0], ((1,), (0,)))
             + _dot(aggs[1], Wp1p_ref[1], ((1,), (0,)))
             + _dot(aggs[2], Wp1p_ref[2], ((1,), (0,))))
    feat1 = feat1 * (1.0 / K) + bp1_ref[...]
    x1 = _lrelu(feat1)

    c3 = c1[:M3, :]
    P3 = jnp.concatenate([c1, x1], axis=1)
    SEL3 = sel3_ref[0]
    GG = _dot(SEL3, P3, ((1,), (0,)))
    RmT = (lax.broadcasted_iota(jnp.int32, (M3 * K, M3), 0) // K
           == lax.broadcasted_iota(jnp.int32, (M3 * K, M3), 1)).astype(f32)
    c3rep = _dot(RmT, c3, ((1,), (0,)))
    rel3 = GG[:, 0:3] - c3rep
    w3 = _relu(_dot(rel3, Ws3_ref[...], ((1,), (0,))) + bs3_ref[...])
    E = (lax.broadcasted_iota(jnp.int32, (NK, NK * 64), 1) // 64
         == lax.broadcasted_iota(jnp.int32, (NK, NK * 64), 0)).astype(f32)
    w_exp = _dot(w3, E, ((1,), (0,)))
    x_tile = jnp.concatenate([GG[:, 3:67]] * NK, axis=1)
    prod3 = w_exp * x_tile
    agg3 = jnp.sum(prod3.reshape(M3, K, NK * 64), axis=1)
    feat3 = _dot(agg3, Wp3_ref[...], ((1,), (0,))) * (1.0 / K) + bp3_ref[...]
    x3 = _lrelu(feat3)

    rel4 = c3 - c3[0:1, :]
    w4 = _relu(_dot(rel4, Ws4_ref[...], ((1,), (0,))) + bs4_ref[...])
    agg4 = _dot(w4, x3, ((0,), (0,))) * (1.0 / K)
    out_ref[0] = agg4


def _head_kernel(agg4_ref, Wp4_ref, bp4_ref, Wout_ref, Wreg_ref,
                 xout_ref, xreg_ref):
    v = _dot(agg4_ref[...], Wp4_ref[...], ((1,), (0,))) + bp4_ref[...]
    mu = jnp.mean(v, axis=0, keepdims=True)
    var = jnp.mean((v - mu) ** 2, axis=0, keepdims=True)
    vn = (v - mu) / jnp.sqrt(var + 1e-5)
    x4 = _lrelu(vn)
    xout = _dot(x4, Wout_ref[...], ((1,), (0,)))
    xout_ref[...] = xout
    xreg_ref[...] = _dot(_lrelu(xout), Wreg_ref[...], ((1,), (0,)))


def kernel(x, input_pts, Ws1, bs1, Wp1, bp1, Ws3, bs3, Wp3, bp3,
           Ws4, bs4, Wp4, bp4, Wout, Wreg):
    f32 = jnp.float32
    Wp1p = Wp1.reshape(NK, 3, 64).transpose(1, 0, 2)
    payload = jnp.concatenate(
        [input_pts, x, jnp.zeros((B, N, 10), f32)], axis=-1)
    payload = payload.reshape(B, N * 16)
    ptsT = jnp.transpose(input_pts, (0, 2, 1))

    full = lambda s: pl.BlockSpec(s, lambda b: (0,) * len(s))
    whole = lambda s: pl.BlockSpec(s, lambda: (0,) * len(s))

    Bh = B // 2
    sc_half = _make_sc_select(Bh, B // Bh)

    def run_d2(ptsh, ptsTh):
        return pl.pallas_call(
            _d2_kernel,
            grid=(Bh,),
            in_specs=[pl.BlockSpec((1, N, 3), lambda b: (b, 0, 0)),
                      pl.BlockSpec((1, 3, N), lambda b: (b, 0, 0))],
            out_specs=pl.BlockSpec((1, M1, N), lambda b: (b, 0, 0)),
            out_shape=jax.ShapeDtypeStruct((Bh, M1, N), f32),
        )(ptsh, ptsTh)

    def run_dense(Gh, SEL3h, ptsh):
        return pl.pallas_call(
            _dense_kernel,
            grid=(Bh,),
            in_specs=[
                pl.BlockSpec((1, M1, K * 16), lambda b: (b, 0, 0)),
                pl.BlockSpec((1, M3 * K, M1), lambda b: (b, 0, 0)),
                pl.BlockSpec((1, N, 3), lambda b: (b, 0, 0)),
                pl.BlockSpec(memory_space=pltpu.SMEM),
                pl.BlockSpec(memory_space=pltpu.SMEM),
                full((3, NK, 64)), full((64,)),
                full((3, NK)), full((NK,)), full((NK * 64, 192)),
                full((192,)), full((3, NK)), full((NK,)),
            ],
            out_specs=pl.BlockSpec((1, NK, 192), lambda b: (b, 0, 0)),
            out_shape=jax.ShapeDtypeStruct((Bh, NK, 192), f32),
        )(Gh, SEL3h, ptsh, Ws1, bs1, Wp1p, bp1, Ws3, bs3, Wp3, bp3,
          Ws4, bs4)

    parts = []
    for q in range(2):
        sl = slice(q * Bh, (q + 1) * Bh)
        d2q = run_d2(input_pts[sl], ptsT[sl])
        Gq, SELq = sc_half(d2q, payload[sl])
        parts.append((sl, Gq, SELq))
    aggs4 = [run_dense(Gq, SELq.reshape(Bh, M3 * K, M1), input_pts[sl])
             for sl, Gq, SELq in parts]
    agg4all = jnp.concatenate(aggs4, axis=0).reshape(B, NK * 192)

    xout, xreg = pl.pallas_call(
        _head_kernel,
        in_specs=[whole((B, NK * 192)), whole((NK * 192, 384)),
                  whole((384,)), whole((384, 256)), whole((256, 20))],
        out_specs=[whole((B, 256)), whole((B, 20))],
        out_shape=(jax.ShapeDtypeStruct((B, 256), f32),
                   jax.ShapeDtypeStruct((B, 20), f32)),
    )(agg4all, Wp4, bp4, Wout, Wreg)
    return (xout, xreg)
